# baseline (speedup 1.0000x reference)
import jax
import jax.numpy as jnp
from jax import lax
from jax.experimental import pallas as pl
from jax.experimental.pallas import tpu as pltpu
from jax.experimental.pallas import tpu_sc as plsc

NC, NS = 2, 16
NW = NC * NS
B = 4096 * 200
D = 32
BPW = B // NW
CHUNK = 1280
NCHUNK = BPW // CHUNK
NBUF = 3
LAG = 2
HALF = CHUNK // 2


def _emb_body(idx_hbm, table_hbm, out_hbm,
              idx0, idx1, idx2, rows0, rows1, rows2,
              sa0, sa1, sa2, sb0, sb1, sb2, so0, so1, so2):
    idx_v = (idx0, idx1, idx2)
    rows_v = (rows0, rows1, rows2)
    sa = (sa0, sa1, sa2)
    sb = (sb0, sb1, sb2)
    so = (so0, so1, so2)
    wid = lax.axis_index("s") * NC + lax.axis_index("c")
    base = wid * BPW

    gat = {}
    sto = {}
    for i in range(NCHUNK + LAG):
        if i < NCHUNK:
            b = i % NBUF
            if i >= NBUF:
                sto[i - NBUF].wait()
            off = base + i * CHUNK
            pltpu.sync_copy(idx_hbm.at[pl.ds(off, CHUNK)], idx_v[b])
            ga = pltpu.async_copy(table_hbm.at[idx_v[b].at[pl.ds(0, HALF)]],
                                  rows_v[b].at[pl.ds(0, HALF)], sa[b])
            gb = pltpu.async_copy(table_hbm.at[idx_v[b].at[pl.ds(HALF, HALF)]],
                                  rows_v[b].at[pl.ds(HALF, HALF)], sb[b])
            gat[i] = (ga, gb)
        j = i - LAG
        if 0 <= j < NCHUNK:
            bj = j % NBUF
            gat[j][0].wait()
            gat[j][1].wait()
            offj = base + j * CHUNK
            sto[j] = pltpu.async_copy(rows_v[bj], out_hbm.at[pl.ds(offj, CHUNK)],
                                      so[bj])
    for j in range(max(0, NCHUNK - NBUF), NCHUNK):
        sto[j].wait()


@jax.jit
def _emb(ids_flat, weight):
    mesh = plsc.VectorSubcoreMesh(core_axis_name="c", subcore_axis_name="s",
                                  num_cores=NC, num_subcores=NS)
    return pl.kernel(
        _emb_body,
        out_type=jax.ShapeDtypeStruct((B, D), jnp.float32),
        mesh=mesh,
        scratch_types=[
            pltpu.VMEM((CHUNK,), jnp.int32),
            pltpu.VMEM((CHUNK,), jnp.int32),
            pltpu.VMEM((CHUNK,), jnp.int32),
            pltpu.VMEM((CHUNK, D), jnp.float32),
            pltpu.VMEM((CHUNK, D), jnp.float32),
            pltpu.VMEM((CHUNK, D), jnp.float32),
            pltpu.SemaphoreType.DMA,
            pltpu.SemaphoreType.DMA,
            pltpu.SemaphoreType.DMA,
            pltpu.SemaphoreType.DMA,
            pltpu.SemaphoreType.DMA,
            pltpu.SemaphoreType.DMA,
            pltpu.SemaphoreType.DMA,
            pltpu.SemaphoreType.DMA,
            pltpu.SemaphoreType.DMA,
        ],
        compiler_params=pltpu.CompilerParams(use_tc_tiling_on_sc=False),
    )(ids_flat, weight)


def kernel(input_ids, weight):
    ids_flat = input_ids.reshape(-1).astype(jnp.int32)
    out = _emb(ids_flat, weight)
    return out.reshape(input_ids.shape[0], input_ids.shape[1], D)


# speedup vs baseline: 1.3479x; 1.3479x over previous
import jax
import jax.numpy as jnp
from jax import lax
from jax.experimental import pallas as pl
from jax.experimental.pallas import tpu as pltpu
from jax.experimental.pallas import tpu_sc as plsc

NC, NS = 2, 16
NW = NC * NS
B = 4096 * 200
D = 32
DP = 128
BPW = B // NW
CHUNK = 1600
NCHUNK = BPW // CHUNK


def _emb_body(idx_hbm, table_hbm, out_hbm, idx_v, staged, sem_g):
    wid = lax.axis_index("s") * NC + lax.axis_index("c")
    base = wid * BPW

    def step(i, c):
        off = base + i * CHUNK
        pltpu.sync_copy(idx_hbm.at[pl.ds(off, CHUNK)], idx_v)
        pltpu.async_copy(table_hbm.at[idx_v], staged, sem_g).wait()
        pltpu.sync_copy(staged, out_hbm.at[pl.ds(off, CHUNK), pl.ds(0, D)])
        return c

    lax.fori_loop(0, NCHUNK, step, 0)


@jax.jit
def _emb(ids_flat, weight):
    mesh = plsc.VectorSubcoreMesh(core_axis_name="c", subcore_axis_name="s",
                                  num_cores=NC, num_subcores=NS)
    return pl.kernel(
        _emb_body,
        out_type=jax.ShapeDtypeStruct((B, DP), jnp.float32),
        mesh=mesh,
        scratch_types=[
            pltpu.VMEM((CHUNK,), jnp.int32),
            pltpu.VMEM((CHUNK, D), jnp.float32),
            pltpu.SemaphoreType.DMA,
        ],
        compiler_params=pltpu.CompilerParams(use_tc_tiling_on_sc=False),
    )(ids_flat, weight)


def kernel(input_ids, weight):
    ids_flat = input_ids.reshape(-1).astype(jnp.int32)
    out = _emb(ids_flat, weight)
    return out[:, :D].reshape(input_ids.shape[0], input_ids.shape[1], D)
